# Initial kernel scaffold; baseline (speedup 1.0000x reference)
import jax
import jax.numpy as jnp
from jax.experimental import pallas as pl


def kernel(x, edge_index, W1, b1, W2, b2):
    raise NotImplementedError("write your pallas kernel here")


# speedup vs baseline: 11.1230x; 11.1230x over previous
import dataclasses
import functools

import jax
import jax.numpy as jnp
from jax import lax
from jax.experimental import pallas as pl
from jax.experimental.pallas import tpu as pltpu
from jax.experimental.pallas import tpu_sc as plsc

CHUNK = 128
NC = 2
NS = 16


def _cdiv(a, b):
    return (a + b - 1) // b


def _rows_per_tile(n):
    rt = _cdiv(_cdiv(n, NS), 8) * 8
    return rt, n - rt * (NS - 1)


def _copy_out_rows(sid, src_sh, dst_hbm_core, n):
    rt, last = _rows_per_tile(n)
    r0 = sid * rt

    @pl.when(sid < NS - 1)
    def _():
        pltpu.sync_copy(src_sh.at[pl.ds(r0, rt)],
                        dst_hbm_core.at[pl.ds(r0, rt)])

    @pl.when(sid == NS - 1)
    def _():
        pltpu.sync_copy(src_sh.at[pl.ds((NS - 1) * rt, last)],
                        dst_hbm_core.at[pl.ds((NS - 1) * rt, last)])


HR = 80


def _make_degrees():
    ept = CPT * CHUNK
    mesh = plsc.VectorSubcoreMesh(core_axis_name="c", subcore_axis_name="s")
    cp = pltpu.CompilerParams()
    if "needs_layout_passes" in pltpu.CompilerParams.__dataclass_fields__:
        cp = dataclasses.replace(cp, needs_layout_passes=False)

    @functools.partial(
        pl.kernel,
        out_type=(jax.ShapeDtypeStruct((NC, NS, HR, CHUNK), jnp.float32),
                  jax.ShapeDtypeStruct((NC, NS, HR, CHUNK), jnp.float32)),
        mesh=mesh,
        compiler_params=cp,
        scratch_types=[pltpu.VMEM((ept,), jnp.int32),
                       pltpu.VMEM((ept,), jnp.int32),
                       pltpu.VMEM((HR, CHUNK), jnp.float32),
                       pltpu.VMEM((HR, CHUNK), jnp.float32),
                       pltpu.SemaphoreType.DMA],
    )
    def deg_kernel(src_hbm, dst_hbm, zeros_hbm, dout_hbm, din_hbm,
                   sidx, didx, h_out, h_in, isem):
        cid = lax.axis_index("c")
        sid = lax.axis_index("s")
        base = (cid * NS + sid) * ept

        pltpu.async_copy(src_hbm.at[pl.ds(base, ept)], sidx, isem).wait()
        pltpu.async_copy(dst_hbm.at[pl.ds(base, ept)], didx, isem).wait()
        pltpu.sync_copy(zeros_hbm.at[pl.ds(0, HR)], h_out)
        pltpu.sync_copy(zeros_hbm.at[pl.ds(0, HR)], h_in)

        ones_v = jnp.ones((16,), jnp.float32)

        @pl.loop(0, ept // 16)
        def _(g):
            sv = sidx[pl.ds(g * 16, 16)]
            plsc.addupdate_scatter(
                h_out, [lax.shift_right_logical(sv, 7), lax.bitwise_and(sv, 127)],
                ones_v)
            dv = didx[pl.ds(g * 16, 16)]
            plsc.addupdate_scatter(
                h_in, [lax.shift_right_logical(dv, 7), lax.bitwise_and(dv, 127)],
                ones_v)

        pltpu.sync_copy(h_out, dout_hbm.at[cid, sid])
        pltpu.sync_copy(h_in, din_hbm.at[cid, sid])

    return deg_kernel


CPT = 80
NRING = 2
BLK = 16
PAD_ROWS = 8


def _make_edge_pass(n, d):
    n_acc = n + PAD_ROWS
    mesh = plsc.VectorSubcoreMesh(core_axis_name="c", subcore_axis_name="s")

    @functools.partial(
        pl.kernel,
        out_type=jax.ShapeDtypeStruct((NC, n, d), jnp.float32),
        mesh=mesh,
        scratch_types=[pltpu.VMEM((2, 1, BLK * CHUNK), jnp.int32),
                       pltpu.VMEM((2, BLK, 1, CHUNK), jnp.int32),
                       pltpu.VMEM((NRING, CHUNK, d), jnp.float32),
                       pltpu.VMEM_SHARED((n_acc, d), jnp.float32),
                       pltpu.SemaphoreType.DMA,
                       pltpu.SemaphoreType.DMA((NRING,)),
                       pltpu.SemaphoreType.DMA((NRING,))],
    )
    def edge_kernel(y_hbm, src3d_hbm, dst3d_hbm, zeros_hbm, out_hbm,
                    sidx, didx, rows, acc, isem, gsem, ssem):
        cid = lax.axis_index("c")
        sid = lax.axis_index("s")
        tile_c0 = (cid * NS + sid) * CPT

        rt, last = _rows_per_tile(n)

        @pl.when(sid < NS - 1)
        def _():
            pltpu.sync_copy(zeros_hbm.at[pl.ds(sid * rt, rt)],
                            acc.at[pl.ds(sid * rt, rt)])

        @pl.when(sid == NS - 1)
        def _():
            pltpu.sync_copy(zeros_hbm.at[pl.ds((NS - 1) * rt, last + PAD_ROWS)],
                            acc.at[pl.ds((NS - 1) * rt, last + PAD_ROWS)])

        plsc.subcore_barrier()

        def gather_desc(k):
            slot = lax.rem(k, NRING)
            buf = lax.rem(lax.div(k, BLK), 2)
            idx_view = sidx.at[buf, 0, pl.ds(lax.rem(k, BLK) * CHUNK, CHUNK)]
            return pltpu.make_async_copy(y_hbm.at[idx_view], rows.at[slot],
                                         gsem.at[slot])

        def scatter_desc(k):
            slot = lax.rem(k, NRING)
            buf = lax.rem(lax.div(k, BLK), 2)
            idx_view = didx.at[buf, lax.rem(k, BLK), 0]
            return pltpu.make_async_copy(rows.at[slot], acc.at[idx_view],
                                         ssem.at[slot])

        @pl.loop(0, CPT + 1)
        def _(i):
            @pl.when((lax.rem(i, BLK) == 0) & (i < CPT))
            def _():
                buf = lax.rem(lax.div(i, BLK), 2)
                c0 = tile_c0 + i
                pltpu.async_copy(src3d_hbm.at[lax.div(c0, BLK)],
                                 sidx.at[buf], isem).wait()
                pltpu.async_copy(dst3d_hbm.at[pl.ds(c0, BLK)],
                                 didx.at[buf], isem).wait()

            @pl.when(i < CPT)
            def _():
                @pl.when(i >= NRING)
                def _():
                    scatter_desc(i - NRING).wait()
                gather_desc(i).start()

            @pl.when(i >= 1)
            def _():
                k = i - 1
                gather_desc(k).wait()
                scatter_desc(k).start(add=True)

        for k in range(CPT - NRING, CPT):
            scatter_desc(k).wait()

        plsc.subcore_barrier()
        _copy_out_rows(sid, acc, out_hbm.at[cid], n)

    return edge_kernel


_BLK = 1000


def _matmul(x, w):
    n, din = x.shape
    dout = w.shape[1]

    def body(x_ref, w_ref, o_ref):
        o_ref[...] = jnp.dot(x_ref[...], w_ref[...],
                             preferred_element_type=jnp.float32)

    return pl.pallas_call(
        body,
        grid=(n // _BLK,),
        in_specs=[pl.BlockSpec((_BLK, din), lambda i: (i, 0)),
                  pl.BlockSpec((din, dout), lambda i: (0, 0))],
        out_specs=pl.BlockSpec((_BLK, dout), lambda i: (i, 0)),
        out_shape=jax.ShapeDtypeStruct((n, dout), jnp.float32),
    )(x, w)


def _deg_view(p, n):
    flat = p.reshape(NC * NS, HR * CHUNK)[:, :n]
    return flat.reshape(NC * NS, n // _BLK, 1, _BLK)


def _deg_spec():
    return pl.BlockSpec((NC * NS, 1, 1, _BLK), lambda i: (0, i, 0, 0))


def _norm_from_partials(p_ref):
    deg = jnp.sum(p_ref[...], axis=(0, 1, 2))
    return lax.rsqrt(jnp.maximum(deg, 1.0))


def _scale_rows(z, dout_p):
    n, d = z.shape

    def body(z_ref, p_ref, o_ref):
        ns = _norm_from_partials(p_ref)
        o_ref[...] = z_ref[...] * ns[:, None]

    return pl.pallas_call(
        body,
        grid=(n // _BLK,),
        in_specs=[pl.BlockSpec((_BLK, d), lambda i: (i, 0)),
                  _deg_spec()],
        out_specs=pl.BlockSpec((_BLK, d), lambda i: (i, 0)),
        out_shape=jax.ShapeDtypeStruct((n, d), jnp.float32),
    )(z, dout_p)


def _mid_layer(agg_p, din_p, dout_p, b1):
    n, d = agg_p.shape[1], agg_p.shape[2]

    def body(a_ref, di_ref, do_ref, b_ref, o_ref):
        agg = a_ref[0] + a_ref[1]
        nd = _norm_from_partials(di_ref)
        ns = _norm_from_partials(do_ref)
        h = jnp.maximum(agg * nd[:, None] + b_ref[...], 0.0)
        o_ref[...] = h * ns[:, None]

    return pl.pallas_call(
        body,
        grid=(n // _BLK,),
        in_specs=[pl.BlockSpec((NC, _BLK, d), lambda i: (0, i, 0)),
                  _deg_spec(),
                  _deg_spec(),
                  pl.BlockSpec((1, d), lambda i: (0, 0))],
        out_specs=pl.BlockSpec((_BLK, d), lambda i: (i, 0)),
        out_shape=jax.ShapeDtypeStruct((n, d), jnp.float32),
    )(agg_p, din_p, dout_p, b1)


def _final_layer(agg_p, din_p, w2, b2):
    n, d = agg_p.shape[1], agg_p.shape[2]
    d2 = w2.shape[1]

    def body(a_ref, di_ref, w_ref, b_ref, o_ref):
        agg = a_ref[0] + a_ref[1]
        nd = _norm_from_partials(di_ref)
        y = jnp.dot(agg, w_ref[...], preferred_element_type=jnp.float32)
        o_ref[...] = y * nd[:, None] + b_ref[...]

    return pl.pallas_call(
        body,
        grid=(n // _BLK,),
        in_specs=[pl.BlockSpec((NC, _BLK, d), lambda i: (0, i, 0)),
                  _deg_spec(),
                  pl.BlockSpec((d, d2), lambda i: (0, 0)),
                  pl.BlockSpec((1, d2), lambda i: (0, 0))],
        out_specs=pl.BlockSpec((_BLK, d2), lambda i: (i, 0)),
        out_shape=jax.ShapeDtypeStruct((n, d2), jnp.float32),
    )(agg_p, din_p, w2, b2)


def kernel(x, edge_index, W1, b1, W2, b2):
    n, _ = x.shape
    e = edge_index.shape[1]
    d_hid = W1.shape[1]
    d_out = W2.shape[1]
    nch = e // CHUNK

    src1d = edge_index[0]
    dst1d = edge_index[1]
    z_hid = jnp.zeros((n + PAD_ROWS, d_hid), jnp.float32)

    nch_pad = NC * NS * CPT
    pad = nch_pad * CHUNK - e
    spread = jnp.arange(pad, dtype=jnp.int32) % PAD_ROWS
    src_edge_pad = jnp.concatenate([src1d, spread])
    src3d_pad = src_edge_pad.reshape(nch_pad // BLK, 1, BLK * CHUNK)
    src_deg_pad = jnp.concatenate([src1d, n + spread])
    dst_pad = jnp.concatenate([dst1d, n + spread])
    dst3d_pad = dst_pad.reshape(nch_pad, 1, CHUNK)

    dout_p, din_p = _make_degrees()(src_deg_pad, dst_pad, z_hid)
    dout_p = _deg_view(dout_p, n)
    din_p = _deg_view(din_p, n)
    z1 = _matmul(x, W1)
    y1 = _scale_rows(z1, dout_p)
    edge_pass = _make_edge_pass(n, d_hid)
    agg1 = edge_pass(y1, src3d_pad, dst3d_pad, z_hid)
    y2 = _mid_layer(agg1, din_p, dout_p, b1.reshape(1, -1))
    agg2 = edge_pass(y2, src3d_pad, dst3d_pad, z_hid)
    del d_out, nch
    return _final_layer(agg2, din_p, W2, b2.reshape(1, -1))
